# baseline (device time: 26601 ns/iter reference)
import jax
import jax.numpy as jnp
from jax import lax
from jax.experimental import pallas as pl
from jax.experimental.pallas import tpu as pltpu

N_DEV = 4
B, SQ, SKV, DH = 2, 256, 256, 64
H_LOC = 4
HD_LOC = H_LOC * DH
D_MODEL = 512


def kernel(x, Wq, K_ext, V_ext, Wo):
    def body(x_ref, wq_ref, k_ref, v_ref, wo_ref, out_ref,
             ctx_ref, send_sems, recv_sems):
        my = lax.axis_index("i")
        left = lax.rem(my + N_DEV - 1, N_DEV)
        right = lax.rem(my + 1, N_DEV)

        barrier_sem = pltpu.get_barrier_semaphore()
        for nbr in (left, right):
            pl.semaphore_signal(
                barrier_sem, inc=1,
                device_id=(nbr,), device_id_type=pl.DeviceIdType.MESH,
            )
        pl.semaphore_wait(barrier_sem, 2)

        qb = lax.broadcasted_iota(jnp.int32, (SQ, SKV), 0) // 64
        kb = lax.broadcasted_iota(jnp.int32, (SQ, SKV), 1) // 64
        mask = (qb == kb) | (kb == 0) | (((qb + kb) % 3) == 0)

        wq_loc = wq_ref[:, pl.ds(my * HD_LOC, HD_LOC)].astype(jnp.bfloat16)
        for b in range(B):
            xb = x_ref[b, :, :].astype(jnp.bfloat16)
            qm = jnp.dot(xb, wq_loc, preferred_element_type=jnp.float32)
            for h in range(H_LOC):
                q = qm[:, h * DH:(h + 1) * DH].astype(jnp.bfloat16)
                k = k_ref[b, :, h, :].astype(jnp.bfloat16)
                s = lax.dot_general(
                    q, k, (((1,), (1,)), ((), ())),
                    preferred_element_type=jnp.float32,
                ) * 0.125
                s = jnp.where(mask, s, -1e9)
                w = jnp.exp(s - jnp.max(s, axis=1, keepdims=True))
                w = w / jnp.sum(w, axis=1, keepdims=True)
                v = v_ref[b, :, h, :].astype(jnp.bfloat16)
                c = jnp.dot(w.astype(jnp.bfloat16), v,
                            preferred_element_type=jnp.float32)
                ctx_ref[0, b, :, h * DH:(h + 1) * DH] = c.astype(jnp.bfloat16)

        wo_loc = wo_ref[pl.ds(my * HD_LOC, HD_LOC), :].astype(jnp.bfloat16)
        for b in range(B):
            out_ref[b, :, :] = jnp.dot(ctx_ref[0, b, :, :], wo_loc,
                                       preferred_element_type=jnp.float32)

        for hop in range(N_DEV - 1):
            rdma = pltpu.make_async_remote_copy(
                src_ref=ctx_ref.at[hop],
                dst_ref=ctx_ref.at[hop + 1],
                send_sem=send_sems.at[hop],
                recv_sem=recv_sems.at[hop],
                device_id=(right,),
                device_id_type=pl.DeviceIdType.MESH,
            )
            rdma.start()
            rdma.wait()

            origin = lax.rem(my + N_DEV - hop - 1, N_DEV)
            wo_o = wo_ref[pl.ds(origin * HD_LOC, HD_LOC), :].astype(jnp.bfloat16)
            for b in range(B):
                out_ref[b, :, :] = out_ref[b, :, :] + jnp.dot(
                    ctx_ref[hop + 1, b, :, :], wo_o,
                    preferred_element_type=jnp.float32,
                )

    return pl.pallas_call(
        body,
        out_shape=jax.ShapeDtypeStruct((B, SQ, D_MODEL), jnp.float32),
        in_specs=[pl.BlockSpec(memory_space=pltpu.VMEM)] * 5,
        out_specs=pl.BlockSpec(memory_space=pltpu.VMEM),
        scratch_shapes=[
            pltpu.VMEM((N_DEV, B, SQ, HD_LOC), jnp.bfloat16),
            pltpu.SemaphoreType.DMA((N_DEV - 1,)),
            pltpu.SemaphoreType.DMA((N_DEV - 1,)),
        ],
        compiler_params=pltpu.CompilerParams(collective_id=0),
    )(x, Wq, K_ext, V_ext, Wo)


# device time: 20372 ns/iter; 1.3058x vs baseline; 1.3058x over previous
import jax
import jax.numpy as jnp
from jax import lax
from jax.experimental import pallas as pl
from jax.experimental.pallas import tpu as pltpu

N_DEV = 4
B, SQ, SKV, DH = 2, 256, 256, 64
H_LOC = 4
HD_LOC = H_LOC * DH
D_MODEL = 512


def kernel(x, Wq, K_ext, V_ext, Wo):
    def body(x_ref, wq_ref, k_ref, v_ref, wo_ref, out_ref,
             ctx_ref, send_sems, recv_sems):
        my = lax.axis_index("i")
        left = lax.rem(my + N_DEV - 1, N_DEV)
        right = lax.rem(my + 1, N_DEV)
        diag = lax.rem(my + 2, N_DEV)

        barrier_sem = pltpu.get_barrier_semaphore()
        for nbr in (left, right, diag):
            pl.semaphore_signal(
                barrier_sem, inc=1,
                device_id=(nbr,), device_id_type=pl.DeviceIdType.MESH,
            )
        pl.semaphore_wait(barrier_sem, 3)

        qb = lax.broadcasted_iota(jnp.int32, (SQ, SKV), 0) // 64
        kb = lax.broadcasted_iota(jnp.int32, (SQ, SKV), 1) // 64
        mask = (qb == kb) | (kb == 0) | (((qb + kb) % 3) == 0)

        wq_loc = wq_ref[:, pl.ds(my * HD_LOC, HD_LOC)].astype(jnp.bfloat16)
        for b in range(B):
            xb = x_ref[b, :, :].astype(jnp.bfloat16)
            qm = jnp.dot(xb, wq_loc, preferred_element_type=jnp.float32)
            for h in range(H_LOC):
                q = qm[:, h * DH:(h + 1) * DH].astype(jnp.bfloat16)
                k = k_ref[b, :, h, :].astype(jnp.bfloat16)
                s = lax.dot_general(
                    q, k, (((1,), (1,)), ((), ())),
                    preferred_element_type=jnp.float32,
                ) * 0.125
                s = jnp.where(mask, s, -1e9)
                w = jnp.exp(s - jnp.max(s, axis=1, keepdims=True))
                w = w / jnp.sum(w, axis=1, keepdims=True)
                v = v_ref[b, :, h, :].astype(jnp.bfloat16)
                c = jnp.dot(w.astype(jnp.bfloat16), v,
                            preferred_element_type=jnp.float32)
                ctx_ref[0, b, :, h * DH:(h + 1) * DH] = c.astype(jnp.bfloat16)

        rdmas = []
        for slot, target in ((1, right), (2, left), (3, diag)):
            rdma = pltpu.make_async_remote_copy(
                src_ref=ctx_ref.at[0],
                dst_ref=ctx_ref.at[slot],
                send_sem=send_sems.at[slot - 1],
                recv_sem=recv_sems.at[slot - 1],
                device_id=(target,),
                device_id_type=pl.DeviceIdType.MESH,
            )
            rdma.start()
            rdmas.append(rdma)

        wo_loc = wo_ref[pl.ds(my * HD_LOC, HD_LOC), :].astype(jnp.bfloat16)
        for b in range(B):
            out_ref[b, :, :] = jnp.dot(ctx_ref[0, b, :, :], wo_loc,
                                       preferred_element_type=jnp.float32)

        for (slot, _), rdma, origin in zip(
            ((1, right), (2, left), (3, diag)), rdmas, (left, right, diag)
        ):
            rdma.wait_recv()
            wo_o = wo_ref[pl.ds(origin * HD_LOC, HD_LOC), :].astype(jnp.bfloat16)
            for b in range(B):
                out_ref[b, :, :] = out_ref[b, :, :] + jnp.dot(
                    ctx_ref[slot, b, :, :], wo_o,
                    preferred_element_type=jnp.float32,
                )

        for rdma in rdmas:
            rdma.wait_send()

    return pl.pallas_call(
        body,
        out_shape=jax.ShapeDtypeStruct((B, SQ, D_MODEL), jnp.float32),
        in_specs=[pl.BlockSpec(memory_space=pltpu.VMEM)] * 5,
        out_specs=pl.BlockSpec(memory_space=pltpu.VMEM),
        scratch_shapes=[
            pltpu.VMEM((N_DEV, B, SQ, HD_LOC), jnp.bfloat16),
            pltpu.SemaphoreType.DMA((N_DEV - 1,)),
            pltpu.SemaphoreType.DMA((N_DEV - 1,)),
        ],
        compiler_params=pltpu.CompilerParams(collective_id=0),
    )(x, Wq, K_ext, V_ext, Wo)


# device time: 18791 ns/iter; 1.4156x vs baseline; 1.0841x over previous
import jax
import jax.numpy as jnp
from jax import lax
from jax.experimental import pallas as pl
from jax.experimental.pallas import tpu as pltpu

N_DEV = 4
B, SQ, SKV, DH = 2, 256, 256, 64
H_LOC = 4
HD_LOC = H_LOC * DH
D_MODEL = 512


def kernel(x, Wq, K_ext, V_ext, Wo):
    def body(x_ref, wq_ref, k_ref, v_ref, wo_ref, out_ref,
             ctx_ref, send_sems, recv_sems):
        my = lax.axis_index("i")
        left = lax.rem(my + N_DEV - 1, N_DEV)
        right = lax.rem(my + 1, N_DEV)
        diag = lax.rem(my + 2, N_DEV)

        barrier_sem = pltpu.get_barrier_semaphore()
        for nbr in (left, right, diag):
            pl.semaphore_signal(
                barrier_sem, inc=1,
                device_id=(nbr,), device_id_type=pl.DeviceIdType.MESH,
            )
        pl.semaphore_wait(barrier_sem, 3)

        qb = lax.broadcasted_iota(jnp.int32, (SQ, SKV), 0) // 64
        kb = lax.broadcasted_iota(jnp.int32, (SQ, SKV), 1) // 64
        mask = (qb == kb) | (kb == 0) | (((qb + kb) % 3) == 0)

        wq_loc = wq_ref[:, pl.ds(my * HD_LOC, HD_LOC)].astype(jnp.bfloat16)
        qms = []
        for b in range(B):
            xb = x_ref[b, :, :].astype(jnp.bfloat16)
            qms.append(jnp.dot(xb, wq_loc, preferred_element_type=jnp.float32))

        rdmas = []
        for h in range(H_LOC):
            for b in range(B):
                q = qms[b][:, h * DH:(h + 1) * DH].astype(jnp.bfloat16)
                k = k_ref[b, :, h, :].astype(jnp.bfloat16)
                s = lax.dot_general(
                    q, k, (((1,), (1,)), ((), ())),
                    preferred_element_type=jnp.float32,
                ) * 0.125
                s = jnp.where(mask, s, -1e9)
                w = jnp.exp(s - jnp.max(s, axis=1, keepdims=True))
                w = w / jnp.sum(w, axis=1, keepdims=True)
                v = v_ref[b, :, h, :].astype(jnp.bfloat16)
                c = jnp.dot(w.astype(jnp.bfloat16), v,
                            preferred_element_type=jnp.float32)
                ctx_ref[0, b, :, h * DH:(h + 1) * DH] = c.astype(jnp.bfloat16)
            if h % 2 == 1:
                p = h // 2
                for slot, target in ((1, right), (2, left), (3, diag)):
                    rdma = pltpu.make_async_remote_copy(
                        src_ref=ctx_ref.at[0, :, :, pl.ds(p * 2 * DH, 2 * DH)],
                        dst_ref=ctx_ref.at[slot, :, :, pl.ds(p * 2 * DH, 2 * DH)],
                        send_sem=send_sems.at[slot - 1, p],
                        recv_sem=recv_sems.at[slot - 1, p],
                        device_id=(target,),
                        device_id_type=pl.DeviceIdType.MESH,
                    )
                    rdma.start()
                    rdmas.append(rdma)

        wo_loc = wo_ref[pl.ds(my * HD_LOC, HD_LOC), :].astype(jnp.bfloat16)
        for b in range(B):
            out_ref[b, :, :] = jnp.dot(ctx_ref[0, b, :, :], wo_loc,
                                       preferred_element_type=jnp.float32)

        for slot, origin in ((1, left), (2, right), (3, diag)):
            for p in range(H_LOC // 2):
                rdmas[3 * p + slot - 1].wait_recv()
            wo_o = wo_ref[pl.ds(origin * HD_LOC, HD_LOC), :].astype(jnp.bfloat16)
            for b in range(B):
                out_ref[b, :, :] = out_ref[b, :, :] + jnp.dot(
                    ctx_ref[slot, b, :, :], wo_o,
                    preferred_element_type=jnp.float32,
                )

        for rdma in rdmas:
            rdma.wait_send()

    return pl.pallas_call(
        body,
        out_shape=jax.ShapeDtypeStruct((B, SQ, D_MODEL), jnp.float32),
        in_specs=[pl.BlockSpec(memory_space=pltpu.VMEM)] * 5,
        out_specs=pl.BlockSpec(memory_space=pltpu.VMEM),
        scratch_shapes=[
            pltpu.VMEM((N_DEV, B, SQ, HD_LOC), jnp.bfloat16),
            pltpu.SemaphoreType.DMA((N_DEV - 1, H_LOC // 2)),
            pltpu.SemaphoreType.DMA((N_DEV - 1, H_LOC // 2)),
        ],
        compiler_params=pltpu.CompilerParams(collective_id=0),
    )(x, Wq, K_ext, V_ext, Wo)
